# Initial kernel scaffold; baseline (speedup 1.0000x reference)
import jax
import jax.numpy as jnp
from jax.experimental import pallas as pl


def kernel(logits):
    raise NotImplementedError("write your pallas kernel here")


# speedup vs baseline: 1.3331x; 1.3331x over previous
import functools

import jax
import jax.numpy as jnp
from jax.experimental import pallas as pl
from jax.experimental.pallas import tpu as pltpu

_TOPK = 10
_IMAX = jnp.iinfo(jnp.int32).max
_MERGE_W = 128


def _body(nsteps, vocab, chunk, x_ref, g_ref, o_ref, bv_ref, bi_ref):
    i = pl.program_id(0)
    rows = x_ref.shape[0]
    lanes = jax.lax.broadcasted_iota(jnp.int32, (rows, chunk), 1)
    col = i * chunk + lanes
    x = jnp.where(col < vocab, x_ref[...], -jnp.inf)

    vs, js = [], []
    cv, ci = x, col
    for _ in range(_TOPK):
        m = jnp.max(cv, axis=1, keepdims=True)
        j = jnp.min(jnp.where(cv == m, ci, _IMAX), axis=1, keepdims=True)
        vs.append(m)
        js.append(j)
        cv = jnp.where(ci == j, -jnp.inf, cv)

    @pl.when(i == 0)
    def _():
        bv_ref[...] = jnp.full(bv_ref.shape, -jnp.inf, jnp.float32)
        bi_ref[...] = jnp.full(bi_ref.shape, _IMAX, jnp.int32)

    lane_w = jax.lax.broadcasted_iota(jnp.int32, (rows, _MERGE_W), 1)
    cv2 = bv_ref[...]
    ci2 = bi_ref[...]
    for t in range(_TOPK):
        sel = lane_w == (_TOPK + t)
        cv2 = jnp.where(sel, vs[t], cv2)
        ci2 = jnp.where(sel, js[t], ci2)
    nv = jnp.full((rows, _MERGE_W), -jnp.inf, jnp.float32)
    ni = jnp.full((rows, _MERGE_W), _IMAX, jnp.int32)
    for t in range(_TOPK):
        m = jnp.max(cv2, axis=1, keepdims=True)
        j = jnp.min(jnp.where(cv2 == m, ci2, _IMAX), axis=1, keepdims=True)
        sel = lane_w == t
        nv = jnp.where(sel, m, nv)
        ni = jnp.where(sel, j, ni)
        cv2 = jnp.where(ci2 == j, -jnp.inf, cv2)
    bv_ref[...] = nv
    bi_ref[...] = ni

    @pl.when(i == nsteps - 1)
    def _():
        m = jnp.max(nv, axis=1, keepdims=True)
        u = jnp.exp(nv - m)
        p = u / jnp.sum(u, axis=1, keepdims=True)
        t_ = jnp.log(p + 1e-20) + g_ref[...]
        tm = jnp.max(t_, axis=1, keepdims=True)
        spos = jnp.min(jnp.where(t_ == tm, lane_w, _IMAX), axis=1,
                       keepdims=True)
        o_ref[...] = jnp.min(jnp.where(lane_w == spos, ni, _IMAX), axis=1,
                             keepdims=True)


def _run(logits, chunk, interpret=False):
    rows, vocab = logits.shape
    nsteps = pl.cdiv(vocab, chunk)
    g = jax.random.gumbel(jax.random.key(42), (rows, _TOPK), jnp.float32)
    gpad = jnp.full((rows, _MERGE_W), -jnp.inf, jnp.float32)
    gpad = gpad.at[:, :_TOPK].set(g)
    return pl.pallas_call(
        functools.partial(_body, nsteps, vocab, chunk),
        grid=(nsteps,),
        in_specs=[
            pl.BlockSpec((rows, chunk), lambda i: (0, i)),
            pl.BlockSpec((rows, _MERGE_W), lambda i: (0, 0)),
        ],
        out_specs=pl.BlockSpec((rows, 1), lambda i: (0, 0)),
        out_shape=jax.ShapeDtypeStruct((rows, 1), jnp.int32),
        scratch_shapes=[
            pltpu.VMEM((rows, _MERGE_W), jnp.float32),
            pltpu.VMEM((rows, _MERGE_W), jnp.int32),
        ],
        interpret=interpret,
    )(logits, gpad)


def kernel(logits):
    return _run(logits, chunk=2048)


# speedup vs baseline: 5.8960x; 4.4227x over previous
import functools

import jax
import jax.numpy as jnp
from jax import lax
from jax.experimental import pallas as pl
from jax.experimental.pallas import tpu as pltpu
from jax.experimental.pallas import tpu_sc as plsc

_TOPK = 10
_NC = 16
_IMAX = jnp.iinfo(jnp.int32).max
_LANES = 16
_CHUNKC = 1024
_NQ = 4
_RT = 8


def _merge16(tv, ti, v, vi):
    sa, ia = plsc.sort_key_val(tv, ti, descending=True)
    sb, ib = plsc.sort_key_val(v, vi, descending=True)
    rb = lax.rev(sb, (0,))
    rib = lax.rev(ib, (0,))
    take = sa >= rb
    return jnp.where(take, sa, rb), jnp.where(take, ia, rib)


def _scan_row_block(buf, r8, tv, ti, thr, colbase, nvreg, lane):
    subs = []
    for s in range(0, nvreg, 16):
        cnt = min(16, nvreg - s)
        acc = None
        for u in range(s, s + cnt):
            v = buf[r8, pl.ds(u * _LANES, _LANES)]
            acc = v if acc is None else jnp.maximum(acc, v)
        subs.append((s, cnt, acc))
    g = subs[0][2]
    for _, _, a in subs[1:]:
        g = jnp.maximum(g, a)
    gmax = jnp.max(g)

    def insert(args):
        tv, ti = args
        for s, cnt, a in subs:
            smax = jnp.max(a)

            def ins_sub(args2, s=s, cnt=cnt):
                tv, ti = args2
                for u in range(s, s + cnt):
                    v = buf[r8, pl.ds(u * _LANES, _LANES)]
                    vmax = jnp.max(v)

                    def ins_vreg(args3, u=u, v=v):
                        tv, ti = args3
                        vi = lane + (colbase + u * _LANES)
                        return _merge16(tv, ti, v, vi)

                    tv, ti = lax.cond(vmax > thr, ins_vreg, lambda x: x,
                                      (tv, ti))
                return tv, ti

            tv, ti = lax.cond(smax > thr, ins_sub, lambda x: x, (tv, ti))
        return tv, ti

    return lax.cond(gmax > thr, insert, lambda x: x, (tv, ti))


def _sc_body(vocab, logits_hbm, outv_hbm, outi_hbm, buf, tailbuf, tvs, tis):
    wid = lax.axis_index("s") * 2 + lax.axis_index("c")
    rt = wid // _NQ
    q = wid % _NQ
    row0 = pl.multiple_of(rt * _RT, 8)
    lane = lax.broadcasted_iota(jnp.int32, (_LANES,), 0)

    nmain = vocab // (_NQ * _CHUNKC)
    tail0 = nmain * _NQ * _CHUNKC
    ntail = vocab - tail0

    for r8 in range(_RT):
        tvs[r8] = jnp.full((_LANES,), -jnp.inf, jnp.float32)
        tis[r8] = jnp.zeros((_LANES,), jnp.int32)

    def chunk_body(c, _):
        colbase = (q * nmain + c) * _CHUNKC
        pltpu.sync_copy(
            logits_hbm.at[pl.ds(row0, _RT),
                          pl.ds(pl.multiple_of(colbase, 128), _CHUNKC)],
            buf)

        def row_body(r8, __):
            tv = tvs[r8]
            ti = tis[r8]
            thr = -jnp.max(-tv)
            tv, ti = _scan_row_block(buf, r8, tv, ti, thr, colbase,
                                     _CHUNKC // _LANES, lane)
            tvs[r8] = tv
            tis[r8] = ti
            return 0

        return lax.fori_loop(0, _RT, row_body, 0)

    lax.fori_loop(0, nmain, chunk_body, 0)

    if ntail:
        pltpu.sync_copy(
            logits_hbm.at[pl.ds(row0, _RT), pl.ds(tail0, ntail)], tailbuf)

        def tail_row(r8, __):
            tv = tvs[r8]
            ti = tis[r8]
            thr = -jnp.max(-tv)
            tv, ti = _scan_row_block(tailbuf, r8, tv, ti, thr, tail0,
                                     ntail // _LANES, lane)
            tvs[r8] = tv
            tis[r8] = ti
            return 0

        lax.fori_loop(0, _RT, tail_row, 0)

    out_off = pl.multiple_of(q * 64 + row0, 8)
    pltpu.sync_copy(tvs, outv_hbm.at[pl.ds(out_off, _RT)])
    pltpu.sync_copy(tis, outi_hbm.at[pl.ds(out_off, _RT)])


def _sample_body(v_ref, i_ref, g_ref, o_ref):
    rows = g_ref.shape[0]
    width = _NQ * _NC
    lane = lax.broadcasted_iota(jnp.int32, (rows, width), 1)
    cv = lax.concatenate(
        [v_ref[q * rows:(q + 1) * rows, :] for q in range(_NQ)], 1)
    ci = lax.concatenate(
        [i_ref[q * rows:(q + 1) * rows, :] for q in range(_NQ)], 1)
    nvv = jnp.full((rows, width), -jnp.inf, jnp.float32)
    nii = jnp.zeros((rows, width), jnp.int32)
    for t in range(_TOPK):
        m = jnp.max(cv, axis=1, keepdims=True)
        j = jnp.min(jnp.where(cv == m, ci, _IMAX), axis=1, keepdims=True)
        sel = lane == t
        nvv = jnp.where(sel, m, nvv)
        nii = jnp.where(sel, j, nii)
        cv = jnp.where(ci == j, -jnp.inf, cv)
    m = jnp.max(nvv, axis=1, keepdims=True)
    u = jnp.exp(nvv - m)
    p = u / jnp.sum(u, axis=1, keepdims=True)
    t_ = jnp.log(p + 1e-20) + g_ref[...]
    tm = jnp.max(t_, axis=1, keepdims=True)
    spos = jnp.min(jnp.where(t_ == tm, lane, _IMAX), axis=1, keepdims=True)
    o_ref[...] = jnp.min(jnp.where(lane == spos, nii, _IMAX), axis=1,
                         keepdims=True)


def kernel(logits):
    rows, vocab = logits.shape

    mesh = plsc.VectorSubcoreMesh(core_axis_name="c", subcore_axis_name="s")
    sc_topk = pl.kernel(
        functools.partial(_sc_body, vocab),
        out_type=[jax.ShapeDtypeStruct((_NQ * rows, _NC), jnp.float32),
                  jax.ShapeDtypeStruct((_NQ * rows, _NC), jnp.int32)],
        mesh=mesh,
        scratch_types=[pltpu.VMEM((_RT, _CHUNKC), jnp.float32),
                       pltpu.VMEM((_RT, 576), jnp.float32),
                       pltpu.VMEM((_RT, _NC), jnp.float32),
                       pltpu.VMEM((_RT, _NC), jnp.int32)],
        compiler_params=pltpu.CompilerParams(needs_layout_passes=False),
    )
    cand_v, cand_i = sc_topk(logits)

    g = jax.random.gumbel(jax.random.key(42), (rows, _TOPK), jnp.float32)
    gpad = jnp.full((rows, _NQ * _NC), -jnp.inf, jnp.float32)
    gpad = gpad.at[:, :_TOPK].set(g)

    return pl.pallas_call(
        _sample_body,
        in_specs=[pl.BlockSpec((_NQ * rows, _NC), lambda: (0, 0)),
                  pl.BlockSpec((_NQ * rows, _NC), lambda: (0, 0)),
                  pl.BlockSpec((rows, _NQ * _NC), lambda: (0, 0))],
        out_specs=pl.BlockSpec((rows, 1), lambda: (0, 0)),
        out_shape=jax.ShapeDtypeStruct((rows, 1), jnp.int32),
    )(cand_v, cand_i, gpad)
